# Initial kernel scaffold; baseline (speedup 1.0000x reference)
import functools

import jax
import jax.numpy as jnp
from jax import lax
from jax.experimental import pallas as pl
from jax.experimental.pallas import tpu as pltpu
from jax.experimental.pallas import tpu_sc as plsc

B = 16384
L = 200
EMB = 32
N = B * L

_info = plsc.get_sparse_core_info()
NC, NS = _info.num_cores, _info.num_subcores
NW = NC * NS
B_PER_W = N // NW
CHUNK = 1024
STEPS = B_PER_W // CHUNK


def _emb_kernel(idx_hbm, table_hbm, out_hbm, idx_v, rows_v, sem):
    wid = lax.axis_index("s") * NC + lax.axis_index("c")
    base = wid * B_PER_W

    def body(i, carry):
        off = base + i * CHUNK
        pltpu.sync_copy(idx_hbm.at[pl.ds(off, CHUNK)], idx_v)
        pltpu.async_copy(table_hbm.at[idx_v], rows_v, sem).wait()
        pltpu.sync_copy(rows_v, out_hbm.at[pl.ds(off, CHUNK)])
        return carry

    lax.fori_loop(0, STEPS, body, 0)


def kernel(x, table):
    idx = x.reshape(N).astype(jnp.int32)
    mesh = plsc.VectorSubcoreMesh(core_axis_name="c", subcore_axis_name="s")
    out = pl.kernel(
        _emb_kernel,
        mesh=mesh,
        out_type=jax.ShapeDtypeStruct((N, EMB), jnp.float32),
        scratch_types=[
            pltpu.VMEM((CHUNK,), jnp.int32),
            pltpu.VMEM((CHUNK, EMB), jnp.float32),
            pltpu.SemaphoreType.DMA,
        ],
    )(idx, table)
    return out.reshape(B, L, EMB)


# speedup vs baseline: 4.8090x; 4.8090x over previous
import functools

import jax
import jax.numpy as jnp
from jax import lax
from jax.experimental import pallas as pl
from jax.experimental.pallas import tpu as pltpu
from jax.experimental.pallas import tpu_sc as plsc

B = 16384
L = 200
EMB = 32
N = B * L

_info = plsc.get_sparse_core_info()
NC, NS = _info.num_cores, _info.num_subcores
NW = NC * NS
B_PER_W = N // NW
CHUNK = 1024
STEPS = B_PER_W // CHUNK


def _emb_kernel(idx_hbm, table_hbm, out_hbm, idx_v, rows_v, sem):
    wid = lax.axis_index("s") * NC + lax.axis_index("c")
    base = wid * B_PER_W

    def body(i, carry):
        off = base + i * CHUNK
        pltpu.sync_copy(idx_hbm.at[pl.ds(off, CHUNK)], idx_v)
        pltpu.async_copy(table_hbm.at[idx_v], rows_v, sem).wait()
        pltpu.sync_copy(rows_v, out_hbm.at[pl.ds(off, CHUNK)])
        return carry

    lax.fori_loop(0, STEPS, body, 0)


def kernel(x, table):
    idx = x.reshape(N).astype(jnp.int32)
    mesh = plsc.VectorSubcoreMesh(core_axis_name="c", subcore_axis_name="s")
    out = pl.kernel(
        _emb_kernel,
        mesh=mesh,
        out_type=jax.ShapeDtypeStruct((N, EMB), jnp.float32),
        scratch_types=[
            pltpu.VMEM((CHUNK,), jnp.int32),
            pltpu.VMEM((CHUNK, EMB), jnp.float32),
            pltpu.SemaphoreType.DMA,
        ],
        compiler_params=pltpu.CompilerParams(use_tc_tiling_on_sc=False),
    )(idx, table)
    return out.reshape(B, L, EMB)


# speedup vs baseline: 5.0477x; 1.0496x over previous
import jax
import jax.numpy as jnp
from jax import lax
from jax.experimental import pallas as pl
from jax.experimental.pallas import tpu as pltpu
from jax.experimental.pallas import tpu_sc as plsc

B = 16384
L = 200
EMB = 32
N = B * L

_info = plsc.get_sparse_core_info()
NC, NS = _info.num_cores, _info.num_subcores
NW = NC * NS
B_PER_W = N // NW
CHUNK = 512
S = B_PER_W // CHUNK
NBUF = 4


def _emb_kernel(idx_hbm, table_hbm, out_hbm, *scratch):
    idx_v = scratch[0:NBUF]
    rows_v = scratch[NBUF:2 * NBUF]
    sem_i = scratch[2 * NBUF:3 * NBUF]
    sem_g = scratch[3 * NBUF:4 * NBUF]
    sem_o = scratch[4 * NBUF:5 * NBUF]

    wid = lax.axis_index("s") * NC + lax.axis_index("c")
    base = wid * B_PER_W

    def idx_copy(c, k):
        return pltpu.make_async_copy(
            idx_hbm.at[pl.ds(base + c * CHUNK, CHUNK)], idx_v[k], sem_i[k])

    def gather_copy(k):
        return pltpu.make_async_copy(table_hbm.at[idx_v[k]], rows_v[k], sem_g[k])

    def wb_copy(c, k):
        return pltpu.make_async_copy(
            rows_v[k], out_hbm.at[pl.ds(base + c * CHUNK, CHUNK)], sem_o[k])

    idx_copy(0, 0).start()
    idx_copy(1, 1).start()

    def body(j, carry):
        for k in range(NBUF):
            c = NBUF * j + k

            @pl.when(c >= 2)
            def _():
                gather_copy((k + 2) % NBUF).wait()
                wb_copy(c - 2, (k + 2) % NBUF).start()

            @pl.when(c + 2 < S)
            def _():
                idx_copy(c + 2, (k + 2) % NBUF).start()

            @pl.when(c >= NBUF)
            def _():
                wb_copy(c - NBUF, k).wait()

            idx_copy(c, k).wait()
            gather_copy(k).start()
        return carry

    lax.fori_loop(0, S // NBUF, body, 0)

    for c in (S - 2, S - 1):
        gather_copy(c % NBUF).wait()
        wb_copy(c, c % NBUF).start()
    for c in range(S - NBUF, S):
        wb_copy(c, c % NBUF).wait()


def kernel(x, table):
    idx = x.reshape(N).astype(jnp.int32)
    mesh = plsc.VectorSubcoreMesh(core_axis_name="c", subcore_axis_name="s")
    out = pl.kernel(
        _emb_kernel,
        mesh=mesh,
        out_type=jax.ShapeDtypeStruct((N, EMB), jnp.float32),
        scratch_types=(
            [pltpu.VMEM((CHUNK,), jnp.int32) for _ in range(NBUF)]
            + [pltpu.VMEM((CHUNK, EMB), jnp.float32) for _ in range(NBUF)]
            + [pltpu.SemaphoreType.DMA for _ in range(3 * NBUF)]
        ),
        compiler_params=pltpu.CompilerParams(use_tc_tiling_on_sc=False),
    )(idx, table)
    return out.reshape(B, L, EMB)
